# baseline (speedup 1.0000x reference)
import functools

import jax
import jax.numpy as jnp
from jax.experimental import pallas as pl
from jax.experimental.pallas import tpu as pltpu

_ROWS_PER_STEP = 256


def _tc_gather(x, table, *, batch, dim):
    n_steps = batch // _ROWS_PER_STEP

    def body(idx_sref, t_hbm, out_ref, sem):
        i = pl.program_id(0)
        base = i * _ROWS_PER_STEP
        for j in range(_ROWS_PER_STEP):
            pltpu.make_async_copy(
                t_hbm.at[pl.ds(idx_sref[base + j], 1)],
                out_ref.at[pl.ds(j, 1)],
                sem,
            ).start()
        for j in range(_ROWS_PER_STEP):
            pltpu.make_async_copy(
                t_hbm.at[pl.ds(0, 1)],
                out_ref.at[pl.ds(0, 1)],
                sem,
            ).wait()

    grid_spec = pltpu.PrefetchScalarGridSpec(
        num_scalar_prefetch=1,
        grid=(n_steps,),
        in_specs=[pl.BlockSpec(memory_space=pl.ANY)],
        out_specs=pl.BlockSpec(
            (_ROWS_PER_STEP, dim), lambda i, idx: (i, 0)
        ),
        scratch_shapes=[pltpu.SemaphoreType.DMA],
    )
    return pl.pallas_call(
        body,
        grid_spec=grid_spec,
        out_shape=jax.ShapeDtypeStruct((batch, dim), jnp.float32),
    )(x, table)


def kernel(x_user, x_item, table_user, table_item):
    batch = x_user.shape[0]
    dim = table_user.shape[1]
    xu = x_user.astype(jnp.int32)
    xi = x_item.astype(jnp.int32)
    yu = _tc_gather(xu, table_user, batch=batch, dim=dim)
    yi = _tc_gather(xi, table_item, batch=batch, dim=dim)
    return (yu, yi)


# speedup vs baseline: 1.2183x; 1.2183x over previous
import functools

import jax
import jax.numpy as jnp
from jax import lax
from jax.experimental import pallas as pl
from jax.experimental.pallas import tpu as pltpu
from jax.experimental.pallas import tpu_sc as plsc

_CH = 32
_TC_STEP = 256
_N_TC = 13312


def _tc_gather(x, table, *, rows, dim):
    n_steps = rows // _TC_STEP

    def body(idx_sref, t_hbm, out_ref, sem):
        i = pl.program_id(0)
        base = i * _TC_STEP
        for j in range(_TC_STEP):
            pltpu.make_async_copy(
                t_hbm.at[pl.ds(idx_sref[base + j], 1)],
                out_ref.at[pl.ds(j, 1)],
                sem,
            ).start()
        for j in range(_TC_STEP):
            pltpu.make_async_copy(
                t_hbm.at[pl.ds(0, 1)],
                out_ref.at[pl.ds(0, 1)],
                sem,
            ).wait()

    grid_spec = pltpu.PrefetchScalarGridSpec(
        num_scalar_prefetch=1,
        grid=(n_steps,),
        in_specs=[pl.BlockSpec(memory_space=pl.ANY)],
        out_specs=pl.BlockSpec((_TC_STEP, dim), lambda i, idx: (i, 0)),
        scratch_shapes=[pltpu.SemaphoreType.DMA],
    )
    return pl.pallas_call(
        body,
        grid_spec=grid_spec,
        out_shape=jax.ShapeDtypeStruct((rows, dim), jnp.float32),
    )(x, table)


def _sc_gather(xu2, xi2, tu, ti, *, bu, bi, dim):
    info = plsc.get_sparse_core_info()
    n_workers = info.num_cores * info.num_subcores
    bu_w = bu // n_workers
    bi_w = bi // n_workers

    mesh = plsc.VectorSubcoreMesh(core_axis_name="c", subcore_axis_name="s")

    @functools.partial(
        pl.kernel,
        mesh=mesh,
        out_type=(
            jax.ShapeDtypeStruct((bu, dim), jnp.float32),
            jax.ShapeDtypeStruct((bi, dim), jnp.float32),
        ),
        scratch_types=[
            pltpu.VMEM((bu_w,), jnp.int32),
            pltpu.VMEM((bi_w,), jnp.int32),
            pltpu.VMEM((bi_w, dim), jnp.float32),
            pltpu.SemaphoreType.DMA,
        ],
    )
    def k(xu_hbm, xi_hbm, tu_hbm, ti_hbm, yu_hbm, yi_hbm,
          xu_v, xi_v, rows_v, sem):
        wid = lax.axis_index("s") * info.num_cores + lax.axis_index("c")

        pltpu.async_copy(xu_hbm.at[wid], xu_v, sem).wait()
        pltpu.async_copy(xi_hbm.at[wid], xi_v, sem).wait()

        def fire(t_hbm, x_v, n_rows):
            def body(c, carry):
                off = c * _CH
                for g in range(_CH // 16):
                    vec = x_v[pl.ds(off + g * 16, 16)]
                    for l in range(16):
                        pltpu.async_copy(
                            t_hbm.at[pl.ds(vec[l], 1)],
                            rows_v.at[pl.ds(off + g * 16 + l, 1)],
                            sem,
                        )
                return carry
            lax.fori_loop(0, n_rows // _CH, body, 0)

        def drain_and_store(y_hbm, n_rows):
            base = wid * n_rows
            pltpu.make_async_copy(
                y_hbm.at[pl.ds(base, n_rows)],
                rows_v.at[pl.ds(0, n_rows)],
                sem,
            ).wait()
            pltpu.async_copy(
                rows_v.at[pl.ds(0, n_rows)],
                y_hbm.at[pl.ds(base, n_rows)],
                sem,
            ).wait()

        fire(tu_hbm, xu_v, bu_w)
        drain_and_store(yu_hbm, bu_w)
        fire(ti_hbm, xi_v, bi_w)
        drain_and_store(yi_hbm, bi_w)

    return k(xu2, xi2, tu, ti)


def kernel(x_user, x_item, table_user, table_item):
    batch = x_user.shape[0]
    dim = table_user.shape[1]
    info = plsc.get_sparse_core_info()
    n_workers = info.num_cores * info.num_subcores

    xu = x_user.astype(jnp.int32)
    xi = x_item.astype(jnp.int32)

    bu_sc = batch - _N_TC
    yu_tc = _tc_gather(xu[: _N_TC], table_user, rows=_N_TC, dim=dim)
    yu_sc, yi = _sc_gather(
        xu[_N_TC:].reshape(n_workers, bu_sc // n_workers),
        xi.reshape(n_workers, batch // n_workers),
        table_user, table_item,
        bu=bu_sc, bi=batch, dim=dim,
    )
    return (jnp.concatenate([yu_tc, yu_sc], axis=0), yi)
